# baseline (speedup 1.0000x reference)
import jax
import jax.numpy as jnp
from jax.experimental import pallas as pl
from jax.experimental.pallas import tpu as pltpu

_ALPHA = 0.2
_EPS = 1e-5


def _leaky(v):
    return jnp.maximum(v, _ALPHA * v)


def _elu(v):
    return jnp.where(v > 0, v, jnp.exp(v) - 1.0)


def _attn_layer_batch(hs, W, a_src, a_dst_row):
    Whs = [jnp.dot(h, W, preferred_element_type=jnp.float32) for h in hs]
    f1s = [jnp.dot(Wh, a_src, preferred_element_type=jnp.float32)
           for Wh in Whs]
    f2s = [jax.lax.dot_general(a_dst_row, Wh, (((1,), (1,)), ((), ())),
                               preferred_element_type=jnp.float32)
           for Wh in Whs]
    ms = [_leaky(f1 + jnp.max(f2, axis=1, keepdims=True))
          for f1, f2 in zip(f1s, f2s)]
    E1s = [jnp.exp(f1 - m) for f1, m in zip(f1s, ms)]
    E2s = [jnp.exp(_ALPHA * f1 - m) for f1, m in zip(f1s, ms)]
    R1s = [jnp.exp(f2) for f2 in f2s]
    R2s = [jnp.exp(_ALPHA * f2) for f2 in f2s]
    ps = [jnp.maximum(E1 * R1, E2 * R2)
          for E1, E2, R1, R2 in zip(E1s, E2s, R1s, R2s)]
    attns = [p * (1.0 / jnp.sum(p, axis=1, keepdims=True)) for p in ps]
    return [jnp.dot(attn, Wh, preferred_element_type=jnp.float32)
            for attn, Wh in zip(attns, Whs)]


_G = 16


def _gat_kernel(x_ref, W1_ref, a1s_ref, a1d_ref, W2_ref, a2s_ref, a2d_ref,
                W3_ref, a3s_ref, a3d_ref, out_ref):
    h = [x_ref[g] for g in range(_G)]
    h = [_elu(a) for a in _attn_layer_batch(h, W1_ref[...], a1s_ref[...],
                                            a1d_ref[...])]
    h = [_elu(a) for a in _attn_layer_batch(h, W2_ref[...], a2s_ref[...],
                                            a2d_ref[...])]
    h3 = _attn_layer_batch(h, W3_ref[...], a3s_ref[...], a3d_ref[...])
    for g in range(_G):
        out_ref[g] = h3[g]


def _head_kernel(h_ref, bn1g_ref, bn1b_ref, fc1W_ref, fc1b_ref,
                 bn2g_ref, bn2b_ref, fc2W_ref, fc2b_ref,
                 bn3g_ref, bn3b_ref, fc3W_ref, fc3b_ref,
                 of_ref, out_ref):
    def bn_relu(z, g, b):
        mu = jnp.mean(z, axis=0, keepdims=True)
        var = jnp.mean((z - mu) * (z - mu), axis=0, keepdims=True)
        zn = g * (z - mu) * jax.lax.rsqrt(var + _EPS) + b
        return jnp.maximum(zn, 0.0)

    z = bn_relu(h_ref[...], bn1g_ref[...], bn1b_ref[...])
    z = jnp.dot(z, fc1W_ref[...], preferred_element_type=jnp.float32) + fc1b_ref[...]
    of = bn_relu(z, bn2g_ref[...], bn2b_ref[...])
    of_ref[...] = of
    z = jnp.dot(of, fc2W_ref[...], preferred_element_type=jnp.float32) + fc2b_ref[...]
    z = bn_relu(z, bn3g_ref[...], bn3b_ref[...])
    out_ref[...] = jnp.dot(z, fc3W_ref[...], preferred_element_type=jnp.float32) + fc3b_ref[...]


def kernel(x, W1, a1, W2, a2, W3, a3, bn1_g, bn1_b, fc1_W, fc1_b,
           bn2_g, bn2_b, fc2_W, fc2_b, bn3_g, bn3_b, fc3_W, fc3_b):
    B, N, Fin = x.shape
    H1 = W1.shape[1]
    H2 = W2.shape[1]
    Fo = W3.shape[1]

    a1s, a1d = a1[:H1], a1[H1:].reshape(1, H1)
    a2s, a2d = a2[:H2], a2[H2:].reshape(1, H2)
    a3s, a3d = a3[:Fo], a3[Fo:].reshape(1, Fo)

    rep = lambda shape: pl.BlockSpec(shape, lambda b: (0,) * len(shape))

    h3 = pl.pallas_call(
        _gat_kernel,
        grid=(B // _G,),
        in_specs=[
            pl.BlockSpec((_G, N, Fin), lambda b: (b, 0, 0)),
            rep((Fin, H1)), rep((H1, 1)), rep((1, H1)),
            rep((H1, H2)), rep((H2, 1)), rep((1, H2)),
            rep((H2, Fo)), rep((Fo, 1)), rep((1, Fo)),
        ],
        out_specs=pl.BlockSpec((_G, N, Fo), lambda b: (b, 0, 0)),
        out_shape=jax.ShapeDtypeStruct((B, N, Fo), jnp.float32),
    )(x, W1, a1s, a1d, W2, a2s, a2d, W3, a3s, a3d)

    h3 = h3.reshape(B, N)

    F1 = fc1_W.shape[1]
    F2 = fc2_W.shape[1]
    F3 = fc3_W.shape[1]
    out_feature, out = pl.pallas_call(
        _head_kernel,
        out_shape=(jax.ShapeDtypeStruct((B, F1), jnp.float32),
                   jax.ShapeDtypeStruct((B, F3), jnp.float32)),
    )(h3, bn1_g.reshape(1, N), bn1_b.reshape(1, N),
      fc1_W, fc1_b.reshape(1, F1),
      bn2_g.reshape(1, F1), bn2_b.reshape(1, F1),
      fc2_W, fc2_b.reshape(1, F2),
      bn3_g.reshape(1, F2), bn3_b.reshape(1, F2),
      fc3_W, fc3_b.reshape(1, F3))

    return out_feature, out


# speedup vs baseline: 1.2891x; 1.2891x over previous
import jax
import jax.numpy as jnp
from jax.experimental import pallas as pl
from jax.experimental.pallas import tpu as pltpu

_ALPHA = 0.2
_EPS = 1e-5


def _leaky(v):
    return jnp.maximum(v, _ALPHA * v)


def _elu(v):
    return jnp.where(v > 0, v, jnp.exp(v) - 1.0)


def _attn_tail_batch(WhTs, a_srcT, a_dst):
    f1rs = [jnp.dot(a_srcT, WhT, preferred_element_type=jnp.float32)
            for WhT in WhTs]
    f2cs = [jax.lax.dot_general(WhT, a_dst, (((0,), (0,)), ((), ())),
                                preferred_element_type=jnp.float32)
            for WhT in WhTs]
    f2maxs = [jnp.max(f2c, axis=0, keepdims=True) for f2c in f2cs]
    mrs = [_leaky(f1r + f2max) for f1r, f2max in zip(f1rs, f2maxs)]
    E1rs = [jnp.exp(f1r - mr) for f1r, mr in zip(f1rs, mrs)]
    E2rs = [jnp.exp(_ALPHA * f1r - mr) for f1r, mr in zip(f1rs, mrs)]
    R1cs = [jnp.exp(f2c) for f2c in f2cs]
    R2cs = [jnp.exp(_ALPHA * f2c) for f2c in f2cs]
    pTs = [jnp.maximum(R1c * E1r, R2c * E2r)
           for R1c, E1r, R2c, E2r in zip(R1cs, E1rs, R2cs, E2rs)]
    attnTs = [pT * (1.0 / jnp.sum(pT, axis=0, keepdims=True)) for pT in pTs]
    return [jnp.dot(WhT, attnT, preferred_element_type=jnp.float32)
            for WhT, attnT in zip(WhTs, attnTs)]


_G = 16


def _gat_kernel(x_ref, W1_ref, a1s_ref, a1d_ref, W2n_ref, a2s_ref, a2d_ref,
                W3n_ref, a3s_ref, a3d_ref, out_ref):
    Wh1s = [jnp.dot(x_ref[g], W1_ref[...], preferred_element_type=jnp.float32)
            for g in range(_G)]
    Wh1Ts = [jnp.transpose(Wh1) for Wh1 in Wh1s]
    h = [_elu(a) for a in _attn_tail_batch(Wh1Ts, a1s_ref[...], a1d_ref[...])]
    Wh2Ts = [jnp.dot(W2n_ref[...], hg, preferred_element_type=jnp.float32)
             for hg in h]
    h = [_elu(a) for a in _attn_tail_batch(Wh2Ts, a2s_ref[...], a2d_ref[...])]
    Wh3Ts = [jnp.dot(W3n_ref[...], hg, preferred_element_type=jnp.float32)
             for hg in h]
    h3 = _attn_tail_batch(Wh3Ts, a3s_ref[...], a3d_ref[...])
    for g in range(_G):
        out_ref[g] = h3[g]


def _head_kernel(h_ref, bn1g_ref, bn1b_ref, fc1W_ref, fc1b_ref,
                 bn2g_ref, bn2b_ref, fc2W_ref, fc2b_ref,
                 bn3g_ref, bn3b_ref, fc3W_ref, fc3b_ref,
                 of_ref, out_ref):
    def bn_relu(z, g, b):
        mu = jnp.mean(z, axis=0, keepdims=True)
        var = jnp.mean((z - mu) * (z - mu), axis=0, keepdims=True)
        zn = g * (z - mu) * jax.lax.rsqrt(var + _EPS) + b
        return jnp.maximum(zn, 0.0)

    z = bn_relu(h_ref[...], bn1g_ref[...], bn1b_ref[...])
    z = jnp.dot(z, fc1W_ref[...], preferred_element_type=jnp.float32) + fc1b_ref[...]
    of = bn_relu(z, bn2g_ref[...], bn2b_ref[...])
    of_ref[...] = of
    z = jnp.dot(of, fc2W_ref[...], preferred_element_type=jnp.float32) + fc2b_ref[...]
    z = bn_relu(z, bn3g_ref[...], bn3b_ref[...])
    out_ref[...] = jnp.dot(z, fc3W_ref[...], preferred_element_type=jnp.float32) + fc3b_ref[...]


def kernel(x, W1, a1, W2, a2, W3, a3, bn1_g, bn1_b, fc1_W, fc1_b,
           bn2_g, bn2_b, fc2_W, fc2_b, bn3_g, bn3_b, fc3_W, fc3_b):
    B, N, Fin = x.shape
    H1 = W1.shape[1]
    H2 = W2.shape[1]
    Fo = W3.shape[1]

    W2n = W2.T
    W3n = W3.T
    a1s, a1d = a1[:H1].reshape(1, H1), a1[H1:]
    a2s, a2d = a2[:H2].reshape(1, H2), a2[H2:]
    a3s, a3d = a3[:Fo].reshape(1, Fo), a3[Fo:]

    rep = lambda shape: pl.BlockSpec(shape, lambda b: (0,) * len(shape))

    h3 = pl.pallas_call(
        _gat_kernel,
        grid=(B // _G,),
        in_specs=[
            pl.BlockSpec((_G, N, Fin), lambda b: (b, 0, 0)),
            rep((Fin, H1)), rep((1, H1)), rep((H1, 1)),
            rep((H2, H1)), rep((1, H2)), rep((H2, 1)),
            rep((Fo, H2)), rep((1, Fo)), rep((Fo, 1)),
        ],
        out_specs=pl.BlockSpec((_G, Fo, N), lambda b: (b, 0, 0)),
        out_shape=jax.ShapeDtypeStruct((B, Fo, N), jnp.float32),
    )(x, W1, a1s, a1d, W2n, a2s, a2d, W3n, a3s, a3d)

    h3 = h3.reshape(B, N)

    F1 = fc1_W.shape[1]
    F2 = fc2_W.shape[1]
    F3 = fc3_W.shape[1]
    out_feature, out = pl.pallas_call(
        _head_kernel,
        out_shape=(jax.ShapeDtypeStruct((B, F1), jnp.float32),
                   jax.ShapeDtypeStruct((B, F3), jnp.float32)),
    )(h3, bn1_g.reshape(1, N), bn1_b.reshape(1, N),
      fc1_W, fc1_b.reshape(1, F1),
      bn2_g.reshape(1, F1), bn2_b.reshape(1, F1),
      fc2_W, fc2_b.reshape(1, F2),
      bn3_g.reshape(1, F2), bn3_b.reshape(1, F2),
      fc3_W, fc3_b.reshape(1, F3))

    return out_feature, out
